# Initial kernel scaffold; baseline (speedup 1.0000x reference)
import jax
import jax.numpy as jnp
from jax.experimental import pallas as pl


def kernel(x, table, Wp, Wt0, bt0, Wg0, bg0, Wt1, bt1, Wg1, bg1):
    raise NotImplementedError("write your pallas kernel here")


# speedup vs baseline: 14.5362x; 14.5362x over previous
import functools

import jax
import jax.numpy as jnp
from jax.experimental import pallas as pl
from jax.experimental.pallas import tpu as pltpu
from jax.experimental.pallas import tpu_sc as plsc

V, D, H = 1000000, 32, 64
B, L = 4096, 200
N = B * L

GATHER_WINDOW = 128
MLP_BLOCK = 4096


def _sc_gather(table, idx_flat):
    mesh = plsc.VectorSubcoreMesh(core_axis_name="c", subcore_axis_name="s")
    idx2d = idx_flat.reshape(1, N)

    @functools.partial(
        pl.kernel,
        out_type=jax.ShapeDtypeStruct((N, D), jnp.float32),
        mesh=mesh,
        compiler_params=pltpu.CompilerParams(use_tc_tiling_on_sc=False),
    )
    def gather_kernel(table_hbm, idx_hbm, out_hbm):
        def body(idx_vmem, out_vmem):
            pltpu.sync_copy(table_hbm.at[idx_vmem.at[0]], out_vmem)

        pltpu.emit_pipeline(
            body,
            grid=(N // GATHER_WINDOW,),
            in_specs=[pl.BlockSpec((1, GATHER_WINDOW), lambda i: (0, i))],
            out_specs=[pl.BlockSpec((GATHER_WINDOW, D), lambda i: (i, 0))],
            core_axis_name=("c", "s"),
            dimension_semantics=(pltpu.PARALLEL,),
        )(idx_hbm, out_hbm)

    return gather_kernel(table, idx2d)


def _mlp_body(emb_ref, wp_ref, wt0_ref, bt0_ref, wg0_ref, bg0_ref,
              wt1_ref, bt1_ref, wg1_ref, bg1_ref, out_ref):
    h = jnp.dot(emb_ref[...], wp_ref[...], preferred_element_type=jnp.float32)
    for wt, bt, wg, bg in ((wt0_ref, bt0_ref, wg0_ref, bg0_ref),
                           (wt1_ref, bt1_ref, wg1_ref, bg1_ref)):
        g = jax.nn.sigmoid(
            jnp.dot(h, wg[...], preferred_element_type=jnp.float32) + bg[...])
        t = jnp.maximum(
            jnp.dot(h, wt[...], preferred_element_type=jnp.float32) + bt[...],
            0.0)
        h = g * t + (1.0 - g) * h
    out_ref[...] = h


def _tc_mlp(emb, Wp, Wt0, bt0, Wg0, bg0, Wt1, bt1, Wg1, bg1):
    full = lambda shape: pl.BlockSpec(shape, lambda i: (0, 0))
    return pl.pallas_call(
        _mlp_body,
        grid=(N // MLP_BLOCK,),
        in_specs=[
            pl.BlockSpec((MLP_BLOCK, D), lambda i: (i, 0)),
            full((D, H)),
            full((H, H)), full((1, H)),
            full((H, H)), full((1, H)),
            full((H, H)), full((1, H)),
            full((H, H)), full((1, H)),
        ],
        out_specs=pl.BlockSpec((MLP_BLOCK, H), lambda i: (i, 0)),
        out_shape=jax.ShapeDtypeStruct((N, H), jnp.float32),
        compiler_params=pltpu.CompilerParams(
            dimension_semantics=("parallel",),
        ),
    )(emb, Wp,
      Wt0, bt0.reshape(1, H), Wg0, bg0.reshape(1, H),
      Wt1, bt1.reshape(1, H), Wg1, bg1.reshape(1, H))


def kernel(x, table, Wp, Wt0, bt0, Wg0, bg0, Wt1, bt1, Wg1, bg1):
    idx_flat = x.reshape(N)
    emb = _sc_gather(table, idx_flat)
    h = _tc_mlp(emb, Wp, Wt0, bt0, Wg0, bg0, Wt1, bt1, Wg1, bg1)
    return h.reshape(B, L, H)


# speedup vs baseline: 14.8765x; 1.0234x over previous
import functools

import jax
import jax.numpy as jnp
from jax.experimental import pallas as pl
from jax.experimental.pallas import tpu as pltpu
from jax.experimental.pallas import tpu_sc as plsc

V, D, H = 1000000, 32, 64
B, L = 4096, 200
N = B * L
PACK = 4
NP = N // PACK
DP, HP = D * PACK, H * PACK

GATHER_WINDOW = 128
MLP_BLOCK = 1024


def _sc_gather(table, idx2d):
    mesh = plsc.VectorSubcoreMesh(core_axis_name="c", subcore_axis_name="s")

    @functools.partial(
        pl.kernel,
        out_type=jax.ShapeDtypeStruct((N, D), jnp.float32),
        mesh=mesh,
        compiler_params=pltpu.CompilerParams(use_tc_tiling_on_sc=False),
    )
    def gather_kernel(table_hbm, idx_hbm, out_hbm):
        def body(idx_vmem, out_vmem):
            pltpu.sync_copy(table_hbm.at[idx_vmem.at[0]], out_vmem)

        pltpu.emit_pipeline(
            body,
            grid=(N // GATHER_WINDOW,),
            in_specs=[pl.BlockSpec((1, GATHER_WINDOW), lambda i: (0, i))],
            out_specs=[pl.BlockSpec((GATHER_WINDOW, D), lambda i: (i, 0))],
            core_axis_name=("c", "s"),
            dimension_semantics=(pltpu.PARALLEL,),
        )(idx_hbm, out_hbm)

    return gather_kernel(table, idx2d)


def _mlp_body(emb_ref, wp_ref, wt0_ref, bt0_ref, wg0_ref, bg0_ref,
              wt1_ref, bt1_ref, wg1_ref, bg1_ref, out_ref):
    e = emb_ref[...].astype(jnp.bfloat16)
    h = jnp.dot(e, wp_ref[...], preferred_element_type=jnp.float32)
    for wt, bt, wg, bg in ((wt0_ref, bt0_ref, wg0_ref, bg0_ref),
                           (wt1_ref, bt1_ref, wg1_ref, bg1_ref)):
        hb = h.astype(jnp.bfloat16)
        g = jax.nn.sigmoid(
            jnp.dot(hb, wg[...], preferred_element_type=jnp.float32) + bg[...])
        t = jnp.maximum(
            jnp.dot(hb, wt[...], preferred_element_type=jnp.float32) + bt[...],
            0.0)
        h = g * t + (1.0 - g) * h
    out_ref[...] = h


def _tc_mlp(emb2, Wp2, Wt02, bt02, Wg02, bg02, Wt12, bt12, Wg12, bg12):
    full = lambda shape: pl.BlockSpec(shape, lambda i: (0, 0))
    return pl.pallas_call(
        _mlp_body,
        grid=(NP // MLP_BLOCK,),
        in_specs=[
            pl.BlockSpec((MLP_BLOCK, DP), lambda i: (i, 0)),
            full((DP, HP)),
            full((HP, HP)), full((1, HP)),
            full((HP, HP)), full((1, HP)),
            full((HP, HP)), full((1, HP)),
            full((HP, HP)), full((1, HP)),
        ],
        out_specs=pl.BlockSpec((MLP_BLOCK, HP), lambda i: (i, 0)),
        out_shape=jax.ShapeDtypeStruct((NP, HP), jnp.float32),
        compiler_params=pltpu.CompilerParams(
            dimension_semantics=("parallel",),
        ),
    )(emb2, Wp2, Wt02, bt02, Wg02, bg02, Wt12, bt12, Wg12, bg12)


def _block_diag4(w):
    a, b = w.shape
    out = jnp.zeros((PACK * a, PACK * b), w.dtype)
    for i in range(PACK):
        out = out.at[i * a:(i + 1) * a, i * b:(i + 1) * b].set(w)
    return out.astype(jnp.bfloat16)


def kernel(x, table, Wp, Wt0, bt0, Wg0, bg0, Wt1, bt1, Wg1, bg1):
    idx2d = x.reshape(1, N)
    emb = _sc_gather(table, idx2d)
    emb2 = emb.reshape(NP, DP)

    Wp2 = _block_diag4(Wp)
    args = []
    for wt, bt, wg, bg in ((Wt0, bt0, Wg0, bg0), (Wt1, bt1, Wg1, bg1)):
        args += [_block_diag4(wt), jnp.tile(bt, PACK).reshape(1, HP),
                 _block_diag4(wg), jnp.tile(bg, PACK).reshape(1, HP)]

    h2 = _tc_mlp(emb2, Wp2, *args)
    return h2.reshape(B, L, H)


# speedup vs baseline: 15.3557x; 1.0322x over previous
import functools

import jax
import jax.numpy as jnp
from jax.experimental import pallas as pl
from jax.experimental.pallas import tpu as pltpu
from jax.experimental.pallas import tpu_sc as plsc

V, D, H = 1000000, 32, 64
B, L = 4096, 200
N = B * L
PACK = 4
NP = N // PACK
DP, HP = D * PACK, H * PACK

GATHER_WINDOW = 128
MLP_BLOCK = 1024


def _sc_gather(table, x):
    mesh = plsc.VectorSubcoreMesh(core_axis_name="c", subcore_axis_name="s")
    w0, w1 = 104, 96

    @functools.partial(
        pl.kernel,
        out_type=jax.ShapeDtypeStruct((N, D), jnp.float32),
        mesh=mesh,
        scratch_types=[pltpu.SemaphoreType.DMA],
        compiler_params=pltpu.CompilerParams(use_tc_tiling_on_sc=False),
    )
    def gather_kernel(table_hbm, idx_hbm, out_hbm, sem):
        def body(idx_vmem, out_vmem):
            c0 = pltpu.async_copy(
                table_hbm.at[idx_vmem.at[0, pl.ds(0, w0)]],
                out_vmem.at[pl.ds(0, w0)], sem)
            c1 = pltpu.async_copy(
                table_hbm.at[idx_vmem.at[0, pl.ds(w0, w1)]],
                out_vmem.at[pl.ds(w0, w1)], sem)
            c0.wait()
            c1.wait()

        pltpu.emit_pipeline(
            body,
            grid=(B,),
            in_specs=[pl.BlockSpec((1, L), lambda i: (i, 0))],
            out_specs=[pl.BlockSpec((L, D), lambda i: (i, 0))],
            core_axis_name=("c", "s"),
            dimension_semantics=(pltpu.PARALLEL,),
        )(idx_hbm, out_hbm)

    return gather_kernel(table, x)


def _mlp_body(emb_ref, wp_ref, wt0_ref, bt0_ref, wg0_ref, bg0_ref,
              wt1_ref, bt1_ref, wg1_ref, bg1_ref, out_ref):
    e = emb_ref[...].astype(jnp.bfloat16)
    h = jnp.dot(e, wp_ref[...], preferred_element_type=jnp.float32)
    for wt, bt, wg, bg in ((wt0_ref, bt0_ref, wg0_ref, bg0_ref),
                           (wt1_ref, bt1_ref, wg1_ref, bg1_ref)):
        hb = h.astype(jnp.bfloat16)
        g = jax.nn.sigmoid(
            jnp.dot(hb, wg[...], preferred_element_type=jnp.float32) + bg[...])
        t = jnp.maximum(
            jnp.dot(hb, wt[...], preferred_element_type=jnp.float32) + bt[...],
            0.0)
        h = g * t + (1.0 - g) * h
    out_ref[...] = h


def _tc_mlp(emb2, Wp2, Wt02, bt02, Wg02, bg02, Wt12, bt12, Wg12, bg12):
    full = lambda shape: pl.BlockSpec(shape, lambda i: (0, 0))
    return pl.pallas_call(
        _mlp_body,
        grid=(NP // MLP_BLOCK,),
        in_specs=[
            pl.BlockSpec((MLP_BLOCK, DP), lambda i: (i, 0)),
            full((DP, HP)),
            full((HP, HP)), full((1, HP)),
            full((HP, HP)), full((1, HP)),
            full((HP, HP)), full((1, HP)),
            full((HP, HP)), full((1, HP)),
        ],
        out_specs=pl.BlockSpec((MLP_BLOCK, HP), lambda i: (i, 0)),
        out_shape=jax.ShapeDtypeStruct((NP, HP), jnp.float32),
        compiler_params=pltpu.CompilerParams(
            dimension_semantics=("parallel",),
        ),
    )(emb2, Wp2, Wt02, bt02, Wg02, bg02, Wt12, bt12, Wg12, bg12)


def _block_diag4(w):
    a, b = w.shape
    out = jnp.zeros((PACK * a, PACK * b), w.dtype)
    for i in range(PACK):
        out = out.at[i * a:(i + 1) * a, i * b:(i + 1) * b].set(w)
    return out.astype(jnp.bfloat16)


def kernel(x, table, Wp, Wt0, bt0, Wg0, bg0, Wt1, bt1, Wg1, bg1):
    emb = _sc_gather(table, x)
    emb2 = emb.reshape(NP, DP)

    Wp2 = _block_diag4(Wp)
    args = []
    for wt, bt, wg, bg in ((Wt0, bt0, Wg0, bg0), (Wt1, bt1, Wg1, bg1)):
        args += [_block_diag4(wt), jnp.tile(bt, PACK).reshape(1, HP),
                 _block_diag4(wg), jnp.tile(bg, PACK).reshape(1, HP)]

    h2 = _tc_mlp(emb2, Wp2, *args)
    return h2.reshape(B, L, H)


# speedup vs baseline: 16.0740x; 1.0468x over previous
import dataclasses
import functools

import jax
import jax.numpy as jnp
from jax.experimental import pallas as pl
from jax.experimental.pallas import tpu as pltpu
from jax.experimental.pallas import tpu_sc as plsc

V, D, H = 1000000, 32, 64
B, L = 4096, 200
N = B * L
PACK = 4
NP = N // PACK
DP, HP = D * PACK, H * PACK

GATHER_WINDOW = 128
MLP_BLOCK = 1024


def _sc_gather(table, x):
    mesh = plsc.VectorSubcoreMesh(core_axis_name="c", subcore_axis_name="s")

    n_tiles = 32
    rows_per_tile = B // n_tiles
    XR = 4
    n_steps = rows_per_tile // XR
    TOK = XR * L
    PR = TOK // PACK
    CHUNKS = ((0, 104), (104, 96))

    @functools.partial(
        pl.kernel,
        out_type=jax.ShapeDtypeStruct((NP, DP), jnp.float32),
        mesh=mesh,
        scratch_types=[
            pltpu.VMEM((2, XR, L), jnp.int32),
            pltpu.VMEM((2, TOK, D), jnp.float32),
            pltpu.VMEM((2, PR, DP), jnp.float32),
            pltpu.SemaphoreType.DMA, pltpu.SemaphoreType.DMA,
            pltpu.SemaphoreType.DMA, pltpu.SemaphoreType.DMA,
            pltpu.SemaphoreType.DMA, pltpu.SemaphoreType.DMA,
        ],
        compiler_params=pltpu.CompilerParams(use_tc_tiling_on_sc=False),
    )
    def gather_kernel(table_hbm, idx_hbm, out_hbm, idx_v, rows_v,
                      packed_v, si0, si1, sg0, sg1, so0, so1):
        from jax import lax
        wid = lax.axis_index("s") * 2 + lax.axis_index("c")
        base = wid * rows_per_tile
        sis = (si0, si1)
        sgs = (sg0, sg1)
        sos = (so0, so1)

        def fire_idx_load(i, b):
            pltpu.async_copy(idx_hbm.at[pl.ds(base + i * XR, XR)],
                             idx_v.at[b], sis[b])

        def repack(b):
            @pl.loop(0, PR)
            def _(j):
                for k in range(PACK):
                    for q in range(D // 16):
                        packed_v[b, j, pl.ds(D * k + 16 * q, 16)] = (
                            rows_v[b, 4 * j + k, pl.ds(16 * q, 16)])

        for b in range(2):
            fire_idx_load(b, b)

        @pl.loop(0, n_steps, step=2)
        def _(g):
            copies = [[], []]
            for b in range(2):
                i = g + b
                pltpu.make_async_copy(
                    idx_hbm.at[pl.ds(0, XR)], idx_v.at[b], sis[b]).wait()
                for r in range(XR):
                    for c0, cw in CHUNKS:
                        copies[b].append(pltpu.async_copy(
                            table_hbm.at[idx_v.at[b, r, pl.ds(c0, cw)]],
                            rows_v.at[b, pl.ds(r * L + c0, cw)],
                            sgs[b]))
            for b in range(2):
                i = g + b
                for c in copies[b]:
                    c.wait()
                @pl.when(i >= 2)
                def _():
                    pltpu.make_async_copy(
                        packed_v.at[b], out_hbm.at[pl.ds(0, PR)],
                        sos[b]).wait()
                repack(b)
                pltpu.async_copy(
                    packed_v.at[b],
                    out_hbm.at[pl.ds((base + i * XR) * (L // PACK), PR)],
                    sos[b])
                @pl.when(i + 2 < n_steps)
                def _():
                    fire_idx_load(i + 2, b)

        for b in range(2):
            pltpu.make_async_copy(
                packed_v.at[b], out_hbm.at[pl.ds(0, PR)], sos[b]).wait()

    return gather_kernel(table, x)


def _mlp_body(emb_ref, wp_ref, wt0_ref, bt0_ref, wg0_ref, bg0_ref,
              wt1_ref, bt1_ref, wg1_ref, bg1_ref, out_ref):
    e = emb_ref[...].astype(jnp.bfloat16)
    h = jnp.dot(e, wp_ref[...], preferred_element_type=jnp.float32)
    for wt, bt, wg, bg in ((wt0_ref, bt0_ref, wg0_ref, bg0_ref),
                           (wt1_ref, bt1_ref, wg1_ref, bg1_ref)):
        hb = h.astype(jnp.bfloat16)
        g = jax.nn.sigmoid(
            jnp.dot(hb, wg[...], preferred_element_type=jnp.float32) + bg[...])
        t = jnp.maximum(
            jnp.dot(hb, wt[...], preferred_element_type=jnp.float32) + bt[...],
            0.0)
        h = g * t + (1.0 - g) * h
    out_ref[...] = h


def _tc_mlp(emb2, Wp2, Wt02, bt02, Wg02, bg02, Wt12, bt12, Wg12, bg12):
    full = lambda shape: pl.BlockSpec(shape, lambda i: (0, 0))
    return pl.pallas_call(
        _mlp_body,
        grid=(NP // MLP_BLOCK,),
        in_specs=[
            pl.BlockSpec((MLP_BLOCK, DP), lambda i: (i, 0)),
            full((DP, HP)),
            full((HP, HP)), full((1, HP)),
            full((HP, HP)), full((1, HP)),
            full((HP, HP)), full((1, HP)),
            full((HP, HP)), full((1, HP)),
        ],
        out_specs=pl.BlockSpec((MLP_BLOCK, HP), lambda i: (i, 0)),
        out_shape=jax.ShapeDtypeStruct((NP, HP), jnp.float32),
        compiler_params=pltpu.CompilerParams(
            dimension_semantics=("parallel",),
        ),
    )(emb2, Wp2, Wt02, bt02, Wg02, bg02, Wt12, bt12, Wg12, bg12)


def _block_diag4(w):
    a, b = w.shape
    out = jnp.zeros((PACK * a, PACK * b), w.dtype)
    for i in range(PACK):
        out = out.at[i * a:(i + 1) * a, i * b:(i + 1) * b].set(w)
    return out.astype(jnp.bfloat16)


def kernel(x, table, Wp, Wt0, bt0, Wg0, bg0, Wt1, bt1, Wg1, bg1):
    emb2 = _sc_gather(table, x)

    Wp2 = _block_diag4(Wp)
    args = []
    for wt, bt, wg, bg in ((Wt0, bt0, Wg0, bg0), (Wt1, bt1, Wg1, bg1)):
        args += [_block_diag4(wt), jnp.tile(bt, PACK).reshape(1, HP),
                 _block_diag4(wg), jnp.tile(bg, PACK).reshape(1, HP)]

    h2 = _tc_mlp(emb2, Wp2, *args)
    return h2.reshape(B, L, H)
